# baseline (speedup 1.0000x reference)
import functools

import jax
import jax.numpy as jnp
from jax import lax
from jax.experimental import pallas as pl
from jax.experimental.pallas import tpu as pltpu
from jax.experimental.pallas import tpu_sc as plsc

B, G, A, C = 16, 20, 5000, 20
AP = 5120
SB, LN = 8, 640
HALF = AP // 2
NCHUNK = HALF // 16
THRESHOLD = 0.5
BG = 20
IMG = 224.0
ALPHA = 0.25


def _lane_rot(x, k):
    perm = jnp.bitwise_and(lax.iota(jnp.int32, 16) + (8 >> k), 15).reshape(16, 1)
    dnums = lax.GatherDimensionNumbers(
        offset_dims=(), collapsed_slice_dims=(0,), start_index_map=(0,))
    return lax.gather(x, perm, dnums, (1,),
                      mode=lax.GatherScatterMode.PROMISE_IN_BOUNDS)


def _lane_max_splat(x):
    for k in range(4):
        x = jnp.maximum(x, _lane_rot(x, k))
    return x


def _lane_min_splat(x):
    for k in range(4):
        x = jnp.minimum(x, _lane_rot(x, k))
    return x

@functools.partial(
    pl.kernel,
    out_type=[
        jax.ShapeDtypeStruct((B, AP), jnp.float32),
        jax.ShapeDtypeStruct((B, AP), jnp.float32),
        jax.ShapeDtypeStruct((B, 4, AP), jnp.float32),
    ],
    mesh=plsc.VectorSubcoreMesh(core_axis_name="c", subcore_axis_name="s"),
    scratch_types=[
        pltpu.VMEM((4, HALF), jnp.float32),
        pltpu.VMEM((5, G, 16), jnp.float32),
        pltpu.VMEM((G, 16), jnp.float32),
        pltpu.VMEM((4, G, 16), jnp.float32),
        pltpu.VMEM((HALF,), jnp.float32),
        pltpu.VMEM((HALF,), jnp.int32),
        pltpu.VMEM((HALF,), jnp.float32),
        pltpu.VMEM((2, G, 16), jnp.float32),
        pltpu.VMEM((2, G, 16), jnp.float32),
        pltpu.VMEM((HALF,), jnp.float32),
        pltpu.VMEM((HALF,), jnp.float32),
        pltpu.VMEM((4, HALF), jnp.float32),
        pltpu.VMEM_SHARED((16, 2, G, 16), jnp.float32),
    ],
)
def _sc_match(anch_ref, tgtb_ref, lbl_ref,
              sel_o, cls_o, tgt_o,
              av, tb, lbl, tbn, rm, ra, aar, cmci, pcm, selS, clsS, tgtS, shared):
    c = lax.axis_index("c")
    s = lax.axis_index("s")
    b = c * 8 + s // 2
    h = s % 2
    gbase = h * HALF

    pltpu.sync_copy(anch_ref.at[:, pl.ds(gbase, HALF)], av)
    pltpu.sync_copy(tgtb_ref.at[b], tb.at[pl.ds(0, 4)])
    pltpu.sync_copy(lbl_ref.at[b], lbl)

    for g in range(G):
        tb[4, g] = (tb[2, g] - tb[0, g]) * (tb[3, g] - tb[1, g])
        for cc in range(4):
            tbn[cc, g] = tb[cc, g] / IMG

    def bodya(i, carry):
        off = i * 16
        ax0 = av[0, pl.ds(off, 16)]
        ay0 = av[1, pl.ds(off, 16)]
        ax1 = av[2, pl.ds(off, 16)]
        ay1 = av[3, pl.ds(off, 16)]
        aar[pl.ds(off, 16)] = (ax1 - ax0) * (ay1 - ay0)
        return carry

    lax.fori_loop(0, NCHUNK, bodya, 0, unroll=False)

    for g in range(G):
        bx0 = tb[0, g]
        by0 = tb[1, g]
        bx1 = tb[2, g]
        by1 = tb[3, g]
        ba = tb[4, g]

        def body1(i, colcarry, g=g, bx0=bx0, by0=by0, bx1=bx1, by1=by1, ba=ba):
            cm, ci = colcarry
            off = i * 16
            ax0 = av[0, pl.ds(off, 16)]
            ay0 = av[1, pl.ds(off, 16)]
            ax1 = av[2, pl.ds(off, 16)]
            ay1 = av[3, pl.ds(off, 16)]
            a_area = aar[pl.ds(off, 16)]
            aidx_f = (lax.iota(jnp.int32, 16) + (gbase + off)).astype(jnp.float32)
            ow = jnp.maximum(jnp.minimum(ax1, bx1) - jnp.maximum(bx0, ax0), 0.0)
            oh_ = jnp.maximum(jnp.minimum(ay1, by1) - jnp.maximum(by0, ay0), 0.0)
            ovl = ow * oh_
            iou = ovl / ((ba + a_area) - ovl)
            if g == 0:
                rm[pl.ds(off, 16)] = iou
                ra[pl.ds(off, 16)] = jnp.broadcast_to(jnp.int32(0), (16,))
            else:
                rmv = rm[pl.ds(off, 16)]
                rav = ra[pl.ds(off, 16)]
                upd = iou > rmv
                rm[pl.ds(off, 16)] = jnp.maximum(rmv, iou)
                ra[pl.ds(off, 16)] = jnp.where(upd, g, rav)
            updc = iou > cm
            return (jnp.maximum(cm, iou), jnp.where(updc, aidx_f, ci))

        cm0 = jnp.broadcast_to(jnp.float32(-1.0), (16,))
        ci0 = jnp.broadcast_to(jnp.float32(1e9), (16,))
        cmf, cif = lax.fori_loop(0, NCHUNK, body1, (cm0, ci0), unroll=False)
        cmci[0, g] = cmf
        cmci[1, g] = cif

    pltpu.sync_copy(cmci, shared.at[s])
    plsc.subcore_barrier()
    pltpu.sync_copy(shared.at[s ^ 1], pcm)

    hlow = h == 0
    bidx = []
    for g in range(G):
        m0 = cmci[0, g]
        i0 = cmci[1, g]
        m1 = pcm[0, g]
        i1 = pcm[1, g]
        lm = jnp.where(hlow, m0, m1)
        li = jnp.where(hlow, i0, i1)
        hm = jnp.where(hlow, m1, m0)
        hi = jnp.where(hlow, i1, i0)
        upd2 = hm > lm
        mm = jnp.maximum(lm, hm)
        mi = jnp.where(upd2, hi, li)
        mx = _lane_max_splat(mm)
        cand = jnp.where(mm == mx, mi, 1e9)
        bidx.append(_lane_min_splat(cand))

    def body2(i, carry):
        off = i * 16
        rmv = rm[pl.ds(off, 16)]
        rav = ra[pl.ds(off, 16)]
        aidx_f = (lax.iota(jnp.int32, 16) + (gbase + off)).astype(jnp.float32)
        forced = aidx_f == bidx[0]
        for g in range(1, G):
            forced = forced | (aidx_f == bidx[g])
        selv = (rmv > THRESHOLD) | forced
        clsg = jnp.broadcast_to(jnp.float32(0.0), (16,))
        tg = [jnp.broadcast_to(jnp.float32(0.0), (16,)) for _ in range(4)]
        for g in range(G):
            m = rav == g
            clsg = jnp.where(m, lbl[g], clsg)
            for cc in range(4):
                tg[cc] = jnp.where(m, tbn[cc, g], tg[cc])
        selS[pl.ds(off, 16)] = jnp.where(selv, 1.0, 0.0)
        clsS[pl.ds(off, 16)] = jnp.where(selv, clsg, float(BG))
        for cc in range(4):
            tgtS[cc, pl.ds(off, 16)] = tg[cc]
        return carry

    lax.fori_loop(0, NCHUNK, body2, 0, unroll=False)

    pltpu.sync_copy(selS, sel_o.at[b, pl.ds(gbase, HALF)])
    pltpu.sync_copy(clsS, cls_o.at[b, pl.ds(gbase, HALF)])
    pltpu.sync_copy(tgtS, tgt_o.at[b, :, pl.ds(gbase, HALF)])


def _loss_body(anchors_ref, pb_ref, pl_ref, sel_ref, cls_ref, tgt_ref,
               bb_ref, ll_ref):
    b = pl.program_id(0)

    aidx = (lax.broadcasted_iota(jnp.int32, (SB, LN), 0) * LN
            + lax.broadcasted_iota(jnp.int32, (SB, LN), 1))
    valid_f = (aidx < A).astype(jnp.float32)

    sel_f = sel_ref[0]
    cls = cls_ref[0]
    n_sel = jnp.sum(sel_f)

    ax0 = anchors_ref[0]
    ay0 = anchors_ref[1]
    ax1 = anchors_ref[2]
    ay1 = anchors_ref[3]
    axn0, ayn0, axn1, ayn1 = ax0 / IMG, ay0 / IMG, ax1 / IMG, ay1 / IMG
    aw = axn1 - axn0
    ah = ayn1 - ayn0
    acx = axn0 + 0.5 * aw
    acy = ayn0 + 0.5 * ah
    cx = acx + pb_ref[0, 0] * aw
    cy = acy + pb_ref[0, 1] * ah
    w = aw * jnp.exp(pb_ref[0, 2])
    h = ah * jnp.exp(pb_ref[0, 3])
    pred = [cx - 0.5 * w, cy - 0.5 * h, cx + 0.5 * w, cy + 0.5 * h]
    bb_sum = jnp.zeros((SB, LN), jnp.float32)
    for cc in range(4):
        d = pred[cc] - tgt_ref[0, cc]
        ad = jnp.abs(d)
        bb_sum = bb_sum + jnp.where(ad < 1.0, 0.5 * d * d, ad - 0.5) * sel_f
    bb_loss = jnp.sum(bb_sum) / (n_sel * 4.0)

    ll_acc = jnp.zeros((SB, LN), jnp.float32)
    for cc in range(C):
        x = pl_ref[0, cc]
        oh_c = (cls == float(cc)).astype(jnp.float32)
        p = jax.nn.sigmoid(x)
        p_t = p * oh_c + (1.0 - p) * (1.0 - oh_c)
        alpha_t = ALPHA * oh_c + (1.0 - ALPHA) * (1.0 - oh_c)
        focal_w = alpha_t * (1.0 - p_t)
        bce = jnp.maximum(x, 0.0) - x * oh_c + jnp.log1p(jnp.exp(-jnp.abs(x)))
        ll_acc = ll_acc + focal_w * bce * valid_f
    ll_loss = jnp.sum(ll_acc) / float(A * C)

    @pl.when(b == 0)
    def _():
        bb_ref[...] = jnp.zeros((1, 1), jnp.float32)
        ll_ref[...] = jnp.zeros((1, 1), jnp.float32)

    bb_ref[...] = bb_ref[...] + bb_loss
    ll_ref[...] = ll_ref[...] + ll_loss


@jax.jit
def _ssd_loss(anch_cm, tgtb_bc, lbl_bc, anchors_t, pb_t, pl_t):
    sel, cls, tgt = _sc_match(anch_cm, tgtb_bc, lbl_bc)
    sel_r = sel.reshape(B, SB, LN)
    cls_r = cls.reshape(B, SB, LN)
    tgt_r = tgt.reshape(B, 4, SB, LN)
    out = pl.pallas_call(
        _loss_body,
        grid=(B,),
        in_specs=[
            pl.BlockSpec((4, SB, LN), lambda b: (0, 0, 0)),
            pl.BlockSpec((1, 4, SB, LN), lambda b: (b, 0, 0, 0)),
            pl.BlockSpec((1, C, SB, LN), lambda b: (b, 0, 0, 0)),
            pl.BlockSpec((1, SB, LN), lambda b: (b, 0, 0)),
            pl.BlockSpec((1, SB, LN), lambda b: (b, 0, 0)),
            pl.BlockSpec((1, 4, SB, LN), lambda b: (b, 0, 0, 0)),
        ],
        out_specs=[
            pl.BlockSpec((1, 1), lambda b: (0, 0)),
            pl.BlockSpec((1, 1), lambda b: (0, 0)),
        ],
        out_shape=[
            jax.ShapeDtypeStruct((1, 1), jnp.float32),
            jax.ShapeDtypeStruct((1, 1), jnp.float32),
        ],
        compiler_params=pltpu.CompilerParams(
            dimension_semantics=("arbitrary",),
        ),
    )(anchors_t, pb_t, pl_t, sel_r, cls_r, tgt_r)
    return out[0][0, 0], out[1][0, 0]


def kernel(target_bb_batch, target_label_batch, pred_bb_batch, pred_label_batch, anchors):
    anch_cm = jnp.pad(jnp.transpose(anchors, (1, 0)), ((0, 0), (0, AP - A)))
    tbt = jnp.transpose(target_bb_batch, (0, 2, 1))
    tgtb_bc = jnp.broadcast_to(tbt[:, :, :, None], (B, 4, G, 16))
    lbl_bc = jnp.broadcast_to(
        target_label_batch.astype(jnp.float32)[:, :, None], (B, G, 16))
    anchors_t = anch_cm.reshape(4, SB, LN)
    pb_t = jnp.pad(jnp.transpose(pred_bb_batch, (0, 2, 1)),
                   ((0, 0), (0, 0), (0, AP - A))).reshape(B, 4, SB, LN)
    pl_t = jnp.pad(jnp.transpose(pred_label_batch, (0, 2, 1))[:, :C, :],
                   ((0, 0), (0, 0), (0, AP - A))).reshape(B, C, SB, LN)
    return _ssd_loss(anch_cm, tgtb_bc, lbl_bc, anchors_t, pb_t, pl_t)


# speedup vs baseline: 1.4786x; 1.4786x over previous
import functools

import jax
import jax.numpy as jnp
from jax import lax
from jax.experimental import pallas as pl
from jax.experimental.pallas import tpu as pltpu
from jax.experimental.pallas import tpu_sc as plsc

B, G, A, C = 16, 20, 5000, 20
AP = 5120
SB, LN = 8, 640
HALF = AP // 2
NCHUNK = HALF // 16
THRESHOLD = 0.5
BG = 20
IMG = 224.0
ALPHA = 0.25


def _lane_rot(x, k):
    perm = jnp.bitwise_and(lax.iota(jnp.int32, 16) + (8 >> k), 15).reshape(16, 1)
    dnums = lax.GatherDimensionNumbers(
        offset_dims=(), collapsed_slice_dims=(0,), start_index_map=(0,))
    return lax.gather(x, perm, dnums, (1,),
                      mode=lax.GatherScatterMode.PROMISE_IN_BOUNDS)


def _lane_max_splat(x):
    for k in range(4):
        x = jnp.maximum(x, _lane_rot(x, k))
    return x


def _lane_min_splat(x):
    for k in range(4):
        x = jnp.minimum(x, _lane_rot(x, k))
    return x

@functools.partial(
    pl.kernel,
    out_type=[
        jax.ShapeDtypeStruct((B, AP), jnp.float32),
        jax.ShapeDtypeStruct((B, AP), jnp.float32),
        jax.ShapeDtypeStruct((B, 4, AP), jnp.float32),
    ],
    mesh=plsc.VectorSubcoreMesh(core_axis_name="c", subcore_axis_name="s"),
    scratch_types=[
        pltpu.VMEM((4, HALF), jnp.float32),
        pltpu.VMEM((5, G, 16), jnp.float32),
        pltpu.VMEM((G, 16), jnp.float32),
        pltpu.VMEM((4, G, 16), jnp.float32),
        pltpu.VMEM((HALF,), jnp.float32),
        pltpu.VMEM((HALF,), jnp.int32),
        pltpu.VMEM((2, G, 16), jnp.float32),
        pltpu.VMEM((2, G, 16), jnp.float32),
        pltpu.VMEM((HALF,), jnp.float32),
        pltpu.VMEM((HALF,), jnp.float32),
        pltpu.VMEM((4, HALF), jnp.float32),
        pltpu.VMEM_SHARED((16, 2, G, 16), jnp.float32),
    ],
)
def _sc_match(anch_ref, tgtb_ref, lbl_ref,
              sel_o, cls_o, tgt_o,
              av, tb, lbl, tbn, rm, ra, cmci, pcm, selS, clsS, tgtS, shared):
    c = lax.axis_index("c")
    s = lax.axis_index("s")
    b = c * 8 + s // 2
    h = s % 2
    gbase = h * HALF

    pltpu.sync_copy(anch_ref.at[:, pl.ds(gbase, HALF)], av)
    pltpu.sync_copy(tgtb_ref.at[b], tb.at[pl.ds(0, 4)])
    pltpu.sync_copy(lbl_ref.at[b], lbl)

    for g in range(G):
        tb[4, g] = (tb[2, g] - tb[0, g]) * (tb[3, g] - tb[1, g])
        for cc in range(4):
            tbn[cc, g] = tb[cc, g] / IMG

    for g in range(G):
        cmci[0, g] = jnp.broadcast_to(jnp.float32(-1.0), (16,))
        cmci[1, g] = jnp.broadcast_to(jnp.float32(1e9), (16,))

    def body1(i, carry):
        off = i * 16
        ax0 = av[0, pl.ds(off, 16)]
        ay0 = av[1, pl.ds(off, 16)]
        ax1 = av[2, pl.ds(off, 16)]
        ay1 = av[3, pl.ds(off, 16)]
        a_area = (ax1 - ax0) * (ay1 - ay0)
        aidx_f = (lax.iota(jnp.int32, 16) + (gbase + off)).astype(jnp.float32)
        rmv = None
        rav = None
        for g in range(G):
            bx0 = tb[0, g]
            by0 = tb[1, g]
            bx1 = tb[2, g]
            by1 = tb[3, g]
            ba = tb[4, g]
            ow = jnp.maximum(jnp.minimum(ax1, bx1) - jnp.maximum(bx0, ax0), 0.0)
            oh_ = jnp.maximum(jnp.minimum(ay1, by1) - jnp.maximum(by0, ay0), 0.0)
            ovl = ow * oh_
            iou = ovl / ((ba + a_area) - ovl)
            if g == 0:
                rmv = iou
                rav = jnp.broadcast_to(jnp.int32(0), (16,))
            else:
                upd = iou > rmv
                rmv = jnp.maximum(rmv, iou)
                rav = jnp.where(upd, g, rav)
            cm = cmci[0, g]
            ci = cmci[1, g]
            updc = iou > cm
            cmci[0, g] = jnp.maximum(cm, iou)
            cmci[1, g] = jnp.where(updc, aidx_f, ci)
        rm[pl.ds(off, 16)] = rmv
        ra[pl.ds(off, 16)] = rav
        return carry

    lax.fori_loop(0, NCHUNK, body1, 0, unroll=2)

    pltpu.sync_copy(cmci, shared.at[s])
    plsc.subcore_barrier()
    pltpu.sync_copy(shared.at[s ^ 1], pcm)

    hlow = h == 0
    bidx = []
    for g in range(G):
        m0 = cmci[0, g]
        i0 = cmci[1, g]
        m1 = pcm[0, g]
        i1 = pcm[1, g]
        lm = jnp.where(hlow, m0, m1)
        li = jnp.where(hlow, i0, i1)
        hm = jnp.where(hlow, m1, m0)
        hi = jnp.where(hlow, i1, i0)
        upd2 = hm > lm
        mm = jnp.maximum(lm, hm)
        mi = jnp.where(upd2, hi, li)
        mx = _lane_max_splat(mm)
        cand = jnp.where(mm == mx, mi, 1e9)
        bidx.append(_lane_min_splat(cand))

    def body2(i, carry):
        off = i * 16
        rmv = rm[pl.ds(off, 16)]
        rav = ra[pl.ds(off, 16)]
        aidx_f = (lax.iota(jnp.int32, 16) + (gbase + off)).astype(jnp.float32)
        forced = aidx_f == bidx[0]
        for g in range(1, G):
            forced = forced | (aidx_f == bidx[g])
        selv = (rmv > THRESHOLD) | forced
        clsg = jnp.broadcast_to(jnp.float32(0.0), (16,))
        tg = [jnp.broadcast_to(jnp.float32(0.0), (16,)) for _ in range(4)]
        for g in range(G):
            m = rav == g
            clsg = jnp.where(m, lbl[g], clsg)
            for cc in range(4):
                tg[cc] = jnp.where(m, tbn[cc, g], tg[cc])
        selS[pl.ds(off, 16)] = jnp.where(selv, 1.0, 0.0)
        clsS[pl.ds(off, 16)] = jnp.where(selv, clsg, float(BG))
        for cc in range(4):
            tgtS[cc, pl.ds(off, 16)] = tg[cc]
        return carry

    lax.fori_loop(0, NCHUNK, body2, 0, unroll=2)

    pltpu.sync_copy(selS, sel_o.at[b, pl.ds(gbase, HALF)])
    pltpu.sync_copy(clsS, cls_o.at[b, pl.ds(gbase, HALF)])
    pltpu.sync_copy(tgtS, tgt_o.at[b, :, pl.ds(gbase, HALF)])


def _loss_body(anchors_ref, pb_ref, pl_ref, sel_ref, cls_ref, tgt_ref,
               bb_ref, ll_ref):
    b = pl.program_id(0)

    aidx = (lax.broadcasted_iota(jnp.int32, (SB, LN), 0) * LN
            + lax.broadcasted_iota(jnp.int32, (SB, LN), 1))
    valid_f = (aidx < A).astype(jnp.float32)

    sel_f = sel_ref[0]
    cls = cls_ref[0]
    n_sel = jnp.sum(sel_f)

    ax0 = anchors_ref[0]
    ay0 = anchors_ref[1]
    ax1 = anchors_ref[2]
    ay1 = anchors_ref[3]
    axn0, ayn0, axn1, ayn1 = ax0 / IMG, ay0 / IMG, ax1 / IMG, ay1 / IMG
    aw = axn1 - axn0
    ah = ayn1 - ayn0
    acx = axn0 + 0.5 * aw
    acy = ayn0 + 0.5 * ah
    cx = acx + pb_ref[0, 0] * aw
    cy = acy + pb_ref[0, 1] * ah
    w = aw * jnp.exp(pb_ref[0, 2])
    h = ah * jnp.exp(pb_ref[0, 3])
    pred = [cx - 0.5 * w, cy - 0.5 * h, cx + 0.5 * w, cy + 0.5 * h]
    bb_sum = jnp.zeros((SB, LN), jnp.float32)
    for cc in range(4):
        d = pred[cc] - tgt_ref[0, cc]
        ad = jnp.abs(d)
        bb_sum = bb_sum + jnp.where(ad < 1.0, 0.5 * d * d, ad - 0.5) * sel_f
    bb_loss = jnp.sum(bb_sum) / (n_sel * 4.0)

    ll_acc = jnp.zeros((SB, LN), jnp.float32)
    for cc in range(C):
        x = pl_ref[0, cc]
        oh_c = (cls == float(cc)).astype(jnp.float32)
        p = jax.nn.sigmoid(x)
        p_t = p * oh_c + (1.0 - p) * (1.0 - oh_c)
        alpha_t = ALPHA * oh_c + (1.0 - ALPHA) * (1.0 - oh_c)
        focal_w = alpha_t * (1.0 - p_t)
        bce = jnp.maximum(x, 0.0) - x * oh_c + jnp.log1p(jnp.exp(-jnp.abs(x)))
        ll_acc = ll_acc + focal_w * bce * valid_f
    ll_loss = jnp.sum(ll_acc) / float(A * C)

    @pl.when(b == 0)
    def _():
        bb_ref[...] = jnp.zeros((1, 1), jnp.float32)
        ll_ref[...] = jnp.zeros((1, 1), jnp.float32)

    bb_ref[...] = bb_ref[...] + bb_loss
    ll_ref[...] = ll_ref[...] + ll_loss


@jax.jit
def _ssd_loss(anch_cm, tgtb_bc, lbl_bc, anchors_t, pb_t, pl_t):
    sel, cls, tgt = _sc_match(anch_cm, tgtb_bc, lbl_bc)
    sel_r = sel.reshape(B, SB, LN)
    cls_r = cls.reshape(B, SB, LN)
    tgt_r = tgt.reshape(B, 4, SB, LN)
    out = pl.pallas_call(
        _loss_body,
        grid=(B,),
        in_specs=[
            pl.BlockSpec((4, SB, LN), lambda b: (0, 0, 0)),
            pl.BlockSpec((1, 4, SB, LN), lambda b: (b, 0, 0, 0)),
            pl.BlockSpec((1, C, SB, LN), lambda b: (b, 0, 0, 0)),
            pl.BlockSpec((1, SB, LN), lambda b: (b, 0, 0)),
            pl.BlockSpec((1, SB, LN), lambda b: (b, 0, 0)),
            pl.BlockSpec((1, 4, SB, LN), lambda b: (b, 0, 0, 0)),
        ],
        out_specs=[
            pl.BlockSpec((1, 1), lambda b: (0, 0)),
            pl.BlockSpec((1, 1), lambda b: (0, 0)),
        ],
        out_shape=[
            jax.ShapeDtypeStruct((1, 1), jnp.float32),
            jax.ShapeDtypeStruct((1, 1), jnp.float32),
        ],
        compiler_params=pltpu.CompilerParams(
            dimension_semantics=("arbitrary",),
        ),
    )(anchors_t, pb_t, pl_t, sel_r, cls_r, tgt_r)
    return out[0][0, 0], out[1][0, 0]


def kernel(target_bb_batch, target_label_batch, pred_bb_batch, pred_label_batch, anchors):
    anch_cm = jnp.pad(jnp.transpose(anchors, (1, 0)), ((0, 0), (0, AP - A)))
    tbt = jnp.transpose(target_bb_batch, (0, 2, 1))
    tgtb_bc = jnp.broadcast_to(tbt[:, :, :, None], (B, 4, G, 16))
    lbl_bc = jnp.broadcast_to(
        target_label_batch.astype(jnp.float32)[:, :, None], (B, G, 16))
    anchors_t = anch_cm.reshape(4, SB, LN)
    pb_t = jnp.pad(jnp.transpose(pred_bb_batch, (0, 2, 1)),
                   ((0, 0), (0, 0), (0, AP - A))).reshape(B, 4, SB, LN)
    pl_t = jnp.pad(jnp.transpose(pred_label_batch, (0, 2, 1))[:, :C, :],
                   ((0, 0), (0, 0), (0, AP - A))).reshape(B, C, SB, LN)
    return _ssd_loss(anch_cm, tgtb_bc, lbl_bc, anchors_t, pb_t, pl_t)


# speedup vs baseline: 2.1021x; 1.4217x over previous
import functools

import jax
import jax.numpy as jnp
from jax import lax
from jax.experimental import pallas as pl
from jax.experimental.pallas import tpu as pltpu
from jax.experimental.pallas import tpu_sc as plsc

B, G, A, C = 16, 20, 5000, 20
AP = 5120
SB, LN = 8, 640
NCH = LN // 16
THRESHOLD = 0.5
BG = 20
IMG = 224.0
ALPHA = 0.25


def _prep_body(targets_ref, anchors_ref, colp_ref, rm_ref, ra_ref):
    ax0 = anchors_ref[0]
    ay0 = anchors_ref[1]
    ax1 = anchors_ref[2]
    ay1 = anchors_ref[3]
    a_area = (ax1 - ax0) * (ay1 - ay0)

    aidx = (lax.broadcasted_iota(jnp.int32, (SB, LN), 0) * LN
            + lax.broadcasted_iota(jnp.int32, (SB, LN), 1)).astype(jnp.float32)

    rowmax = None
    rowarg = None
    for g in range(G):
        bx0 = targets_ref[0, 0, g]
        by0 = targets_ref[0, 1, g]
        bx1 = targets_ref[0, 2, g]
        by1 = targets_ref[0, 3, g]
        b_area = (bx1 - bx0) * (by1 - by0)
        ow = jnp.maximum(jnp.minimum(ax1, bx1) - jnp.maximum(bx0, ax0), 0.0)
        oh = jnp.maximum(jnp.minimum(ay1, by1) - jnp.maximum(by0, ay0), 0.0)
        overlaps = ow * oh
        union = (b_area + a_area) - overlaps
        iou = overlaps / union
        m8 = jnp.max(iou, axis=0, keepdims=True)
        i8 = jnp.min(jnp.where(iou == m8, aidx, 1e9), axis=0, keepdims=True)
        colp_ref[0, 0, g] = m8[0]
        colp_ref[0, 1, g] = i8[0]
        if g == 0:
            rowmax = iou
            rowarg = jnp.zeros_like(iou)
        else:
            upd = iou > rowmax
            rowmax = jnp.maximum(rowmax, iou)
            rowarg = jnp.where(upd, float(g), rowarg)

    rm_ref[0] = rowmax
    ra_ref[0] = rowarg


def _lane_rot(x, k):
    perm = jnp.bitwise_and(lax.iota(jnp.int32, 16) + (8 >> k), 15).reshape(16, 1)
    dnums = lax.GatherDimensionNumbers(
        offset_dims=(), collapsed_slice_dims=(0,), start_index_map=(0,))
    return lax.gather(x, perm, dnums, (1,),
                      mode=lax.GatherScatterMode.PROMISE_IN_BOUNDS)


def _lane_max_splat(x):
    for k in range(4):
        x = jnp.maximum(x, _lane_rot(x, k))
    return x


def _lane_min_splat(x):
    for k in range(4):
        x = jnp.minimum(x, _lane_rot(x, k))
    return x


@functools.partial(
    pl.kernel,
    out_type=[
        jax.ShapeDtypeStruct((B, G, 16), jnp.float32),
    ],
    mesh=plsc.VectorSubcoreMesh(core_axis_name="c", subcore_axis_name="s"),
    scratch_types=[
        pltpu.VMEM((2, G, LN), jnp.float32),
        pltpu.VMEM((G, 16), jnp.float32),
    ],
)
def _sc_match(colp_ref, bidx_o, cps, bidxS):
    c = lax.axis_index("c")
    s = lax.axis_index("s")

    @pl.when(s < 8)
    def _():
        b = c * 8 + s
        pltpu.sync_copy(colp_ref.at[b], cps)

        for g in range(G):
            def body(i, carry, g=g):
                cm, ci = carry
                off = i * 16
                m = cps[0, g, pl.ds(off, 16)]
                idx = cps[1, g, pl.ds(off, 16)]
                upd = m > cm
                return (jnp.maximum(cm, m), jnp.where(upd, idx, ci))

            cm0 = jnp.broadcast_to(jnp.float32(-1.0), (16,))
            ci0 = jnp.broadcast_to(jnp.float32(1e9), (16,))
            cm, ci = lax.fori_loop(0, NCH, body, (cm0, ci0), unroll=False)
            mx = _lane_max_splat(cm)
            bidxS[g] = _lane_min_splat(jnp.where(cm == mx, ci, 1e9))

        pltpu.sync_copy(bidxS, bidx_o.at[b])


def _loss_body(targets_ref, anchors_ref, pb_ref, pl_ref, rm_ref, ra_ref,
               bidx_ref, bb_ref, ll_ref):
    b = pl.program_id(0)

    aidx_i = (lax.broadcasted_iota(jnp.int32, (SB, LN), 0) * LN
              + lax.broadcasted_iota(jnp.int32, (SB, LN), 1))
    aidx = aidx_i.astype(jnp.float32)
    valid_f = (aidx_i < A).astype(jnp.float32)

    is_best = aidx == bidx_ref[0, 0, 0]
    for g in range(1, G):
        is_best = is_best | (aidx == bidx_ref[0, g, 0])
    sel = (rm_ref[0] > THRESHOLD) | is_best
    sel_f = sel.astype(jnp.float32)
    n_sel = jnp.sum(sel_f)

    rowarg = ra_ref[0]
    cls = jnp.zeros((SB, LN), jnp.float32)
    tgt = [jnp.zeros((SB, LN), jnp.float32) for _ in range(4)]
    for g in range(G):
        match = (rowarg == float(g)).astype(jnp.float32)
        cls = cls + match * targets_ref[0, 4, g]
        for cc in range(4):
            tgt[cc] = tgt[cc] + match * (targets_ref[0, cc, g] / IMG)
    cls = jnp.where(sel, cls, float(BG))

    ax0 = anchors_ref[0]
    ay0 = anchors_ref[1]
    ax1 = anchors_ref[2]
    ay1 = anchors_ref[3]
    axn0, ayn0, axn1, ayn1 = ax0 / IMG, ay0 / IMG, ax1 / IMG, ay1 / IMG
    aw = axn1 - axn0
    ah = ayn1 - ayn0
    acx = axn0 + 0.5 * aw
    acy = ayn0 + 0.5 * ah
    cx = acx + pb_ref[0, 0] * aw
    cy = acy + pb_ref[0, 1] * ah
    w = aw * jnp.exp(pb_ref[0, 2])
    h = ah * jnp.exp(pb_ref[0, 3])
    pred = [cx - 0.5 * w, cy - 0.5 * h, cx + 0.5 * w, cy + 0.5 * h]
    bb_sum = jnp.zeros((SB, LN), jnp.float32)
    for cc in range(4):
        d = pred[cc] - tgt[cc]
        ad = jnp.abs(d)
        bb_sum = bb_sum + jnp.where(ad < 1.0, 0.5 * d * d, ad - 0.5) * sel_f
    bb_loss = jnp.sum(bb_sum) / (n_sel * 4.0)

    ll_acc = jnp.zeros((SB, LN), jnp.float32)
    for cc in range(C):
        x = pl_ref[0, cc]
        oh_c = (cls == float(cc)).astype(jnp.float32)
        p = jax.nn.sigmoid(x)
        p_t = p * oh_c + (1.0 - p) * (1.0 - oh_c)
        alpha_t = ALPHA * oh_c + (1.0 - ALPHA) * (1.0 - oh_c)
        focal_w = alpha_t * (1.0 - p_t)
        bce = jnp.maximum(x, 0.0) - x * oh_c + jnp.log1p(jnp.exp(-jnp.abs(x)))
        ll_acc = ll_acc + focal_w * bce * valid_f
    ll_loss = jnp.sum(ll_acc) / float(A * C)

    @pl.when(b == 0)
    def _():
        bb_ref[...] = jnp.zeros((1, 1), jnp.float32)
        ll_ref[...] = jnp.zeros((1, 1), jnp.float32)

    bb_ref[...] = bb_ref[...] + bb_loss
    ll_ref[...] = ll_ref[...] + ll_loss


@jax.jit
def _ssd_loss(targets, anchors_t, pb_t, pl_t):
    colp, rm, ra = pl.pallas_call(
        _prep_body,
        grid=(B,),
        in_specs=[
            pl.BlockSpec((1, 8, G), lambda b: (b, 0, 0)),
            pl.BlockSpec((4, SB, LN), lambda b: (0, 0, 0)),
        ],
        out_specs=[
            pl.BlockSpec((1, 2, G, LN), lambda b: (b, 0, 0, 0)),
            pl.BlockSpec((1, SB, LN), lambda b: (b, 0, 0)),
            pl.BlockSpec((1, SB, LN), lambda b: (b, 0, 0)),
        ],
        out_shape=[
            jax.ShapeDtypeStruct((B, 2, G, LN), jnp.float32),
            jax.ShapeDtypeStruct((B, SB, LN), jnp.float32),
            jax.ShapeDtypeStruct((B, SB, LN), jnp.float32),
        ],
        compiler_params=pltpu.CompilerParams(
            dimension_semantics=("parallel",),
        ),
    )(targets, anchors_t)

    bidx, = _sc_match(colp)

    out = pl.pallas_call(
        _loss_body,
        grid=(B,),
        in_specs=[
            pl.BlockSpec((1, 8, G), lambda b: (b, 0, 0)),
            pl.BlockSpec((4, SB, LN), lambda b: (0, 0, 0)),
            pl.BlockSpec((1, 4, SB, LN), lambda b: (b, 0, 0, 0)),
            pl.BlockSpec((1, C, SB, LN), lambda b: (b, 0, 0, 0)),
            pl.BlockSpec((1, SB, LN), lambda b: (b, 0, 0)),
            pl.BlockSpec((1, SB, LN), lambda b: (b, 0, 0)),
            pl.BlockSpec((1, G, 16), lambda b: (b, 0, 0)),
        ],
        out_specs=[
            pl.BlockSpec((1, 1), lambda b: (0, 0)),
            pl.BlockSpec((1, 1), lambda b: (0, 0)),
        ],
        out_shape=[
            jax.ShapeDtypeStruct((1, 1), jnp.float32),
            jax.ShapeDtypeStruct((1, 1), jnp.float32),
        ],
        compiler_params=pltpu.CompilerParams(
            dimension_semantics=("arbitrary",),
        ),
    )(targets, anchors_t, pb_t, pl_t, rm, ra, bidx)
    return out[0][0, 0], out[1][0, 0]


def kernel(target_bb_batch, target_label_batch, pred_bb_batch, pred_label_batch, anchors):
    targets = jnp.concatenate(
        [jnp.transpose(target_bb_batch, (0, 2, 1)),
         target_label_batch.astype(jnp.float32)[:, None, :],
         jnp.zeros((B, 3, G), jnp.float32)], axis=1)
    anchors_t = jnp.pad(jnp.transpose(anchors, (1, 0)),
                        ((0, 0), (0, AP - A))).reshape(4, SB, LN)
    pb_t = jnp.pad(jnp.transpose(pred_bb_batch, (0, 2, 1)),
                   ((0, 0), (0, 0), (0, AP - A))).reshape(B, 4, SB, LN)
    pl_t = jnp.pad(jnp.transpose(pred_label_batch, (0, 2, 1))[:, :C, :],
                   ((0, 0), (0, 0), (0, AP - A))).reshape(B, C, SB, LN)
    return _ssd_loss(targets, anchors_t, pb_t, pl_t)
